# baseline (speedup 1.0000x reference)
import functools

import jax
import jax.numpy as jnp
from jax import lax
from jax.experimental import pallas as pl
from jax.experimental.pallas import tpu as pltpu
from jax.experimental.pallas import tpu_sc as plsc

N_NODES = 10000
N_EDGES = 320000
D_FEAT = 128
HIDDEN = 32
N_CLASSES = 40

NC = 2
NS = 16
L = 16
HH = HIDDEN // NC

CH = 256
NBUF = 4
HALF_CH = 40
HALF_E = HALF_CH * CH
N_HALVES = 2
T_CH = HALF_CH * N_HALVES
T_EDGES = T_CH * CH
E_PAD = NS * T_EDGES
N_PAD = 10240
ROWS_PER_SUB = N_PAD // NS


def _make_sc_gcn():
    mesh = plsc.VectorSubcoreMesh(core_axis_name="c", subcore_axis_name="s")

    @functools.partial(
        pl.kernel,
        out_type=jax.ShapeDtypeStruct((NC, N_PAD, HH), jnp.float32),
        mesh=mesh,
        compiler_params=pltpu.CompilerParams(use_tc_tiling_on_sc=False),
        scratch_types=[
            pltpu.VMEM((HALF_E,), jnp.int32),
            pltpu.VMEM((HALF_CH, CH), jnp.int32),
            pltpu.VMEM((HALF_E,), jnp.float32),
            [pltpu.VMEM((CH, HH), jnp.float32) for _ in range(NBUF)],
            pltpu.VMEM((1, HH), jnp.float32),
            [pltpu.SemaphoreType.DMA for _ in range(NBUF)],
            pltpu.SemaphoreType.DMA,
            pltpu.VMEM_SHARED((N_PAD, HH), jnp.float32),
            pltpu.VMEM_SHARED((N_PAD, HH), jnp.float32),
        ],
    )
    def sc_gcn(feat_hbm, col_hbm, row_hbm, w_hbm, b1_hbm, out_hbm,
               col_v, row_v, w_v, bufs, b1_v, gsems, ssem, acc1, acc2):
        c = lax.axis_index("c")
        s = lax.axis_index("s")
        row_base = s * ROWS_PER_SUB

        pltpu.sync_copy(feat_hbm.at[c, pl.ds(row_base, ROWS_PER_SUB)],
                        acc2.at[pl.ds(row_base, ROWS_PER_SUB)])
        pltpu.sync_copy(b1_hbm.at[c], b1_v)
        bufs[0][...] = jnp.zeros((CH, HH), jnp.float32)

        def zero_into(dst):
            def zbody(z, carry):
                pltpu.sync_copy(bufs[0], dst.at[pl.ds(row_base + z * CH, CH)])
                return carry
            lax.fori_loop(0, ROWS_PER_SUB // CH, zbody, 0)

        zero_into(acc1)
        plsc.subcore_barrier()

        LOOKAHEAD = NBUF - 1

        def spmm(src, dst):
            def half_body(h, carry):
                ebase = s * T_EDGES + h * HALF_E
                pltpu.sync_copy(col_hbm.at[pl.ds(ebase, HALF_E)], col_v)
                pltpu.sync_copy(w_hbm.at[pl.ds(ebase, HALF_E)], w_v)

                def rstart(i, carry2):
                    pltpu.async_copy(row_hbm.at[pl.ds(ebase + i * CH, CH)],
                                     row_v.at[i], ssem)
                    return carry2
                lax.fori_loop(0, HALF_CH, rstart, 0)

                def rwait(i, carry2):
                    pltpu.make_async_copy(row_hbm.at[pl.ds(ebase + i * CH, CH)],
                                          row_v.at[i], ssem).wait()
                    return carry2
                lax.fori_loop(0, HALF_CH, rwait, 0)

                def gather_start(lc, j):
                    pltpu.async_copy(src.at[col_v.at[pl.ds(lc * CH, CH)]],
                                     bufs[j], gsems[j])

                def gather_wait(lc, j):
                    pltpu.make_async_copy(src.at[col_v.at[pl.ds(lc * CH, CH)]],
                                          bufs[j], gsems[j]).wait()

                for pj in range(LOOKAHEAD):
                    gather_start(pj, pj)

                def body(k, carry2):
                    for j in range(NBUF):
                        lc = k * NBUF + j
                        gather_wait(lc, j)
                        for gq in range(CH // L):
                            wv = w_v[pl.ds(lc * CH + gq * L, L)]
                            for ll in range(L):
                                r = gq * L + ll
                                spl = jnp.broadcast_to(wv[ll], (L,))
                                bufs[j][r, :] = bufs[j][r, :] * spl
                        pltpu.sync_copy(bufs[j], dst.at[row_v.at[lc]],
                                        add=True)
                        lc2 = lc + LOOKAHEAD

                        @pl.when(lc2 < HALF_CH)
                        def _():
                            gather_start(lc2, (j + LOOKAHEAD) % NBUF)
                    return carry2

                lax.fori_loop(0, HALF_CH // NBUF, body, 0)
                return carry

            lax.fori_loop(0, N_HALVES, half_body, 0)

        spmm(acc2, acc1)
        plsc.subcore_barrier()

        b1exp = jnp.broadcast_to(b1_v[...], (CH, HH))

        def relu_body(z, carry):
            sl = pl.ds(row_base + z * CH, CH)
            pltpu.sync_copy(acc1.at[sl], bufs[1])
            bufs[1][...] = jnp.maximum(bufs[1][...] + b1exp, 0.0)
            pltpu.sync_copy(bufs[1], acc1.at[sl])
            return carry

        lax.fori_loop(0, ROWS_PER_SUB // CH, relu_body, 0)
        bufs[0][...] = jnp.zeros((CH, HH), jnp.float32)
        zero_into(acc2)
        plsc.subcore_barrier()

        spmm(acc1, acc2)
        plsc.subcore_barrier()

        pltpu.sync_copy(acc2.at[pl.ds(row_base, ROWS_PER_SUB)],
                        out_hbm.at[c, pl.ds(row_base, ROWS_PER_SUB)])

    return sc_gcn


_sc_gcn = _make_sc_gcn()


_BMX = 2000
_BM = 2048


def _mm1_body(x_ref, w_ref, o_ref):
    o_ref[0] = lax.dot_general(x_ref[...], w_ref[0],
                               (((1,), (1,)), ((), ())),
                               preferred_element_type=jnp.float32)


def _matmul_xw1(x, W1):
    return pl.pallas_call(
        _mm1_body,
        grid=(N_NODES // _BMX, NC),
        in_specs=[
            pl.BlockSpec((_BMX, D_FEAT), lambda i, c: (i, 0)),
            pl.BlockSpec((1, HH, D_FEAT), lambda i, c: (c, 0, 0)),
        ],
        out_specs=pl.BlockSpec((1, _BMX, HH), lambda i, c: (c, i, 0)),
        out_shape=jax.ShapeDtypeStruct((NC, N_PAD, HH), jnp.float32),
    )(x, W1.reshape(NC, HH, D_FEAT))


def _final_body(p_ref, w_ref, b_ref, o_ref):
    h = jnp.concatenate([p_ref[0], p_ref[1]], axis=1)
    o_ref[...] = lax.dot_general(h, w_ref[...], (((1,), (1,)), ((), ())),
                                 preferred_element_type=jnp.float32) + b_ref[...]


def _final(p, W2, b2):
    return pl.pallas_call(
        _final_body,
        grid=(N_PAD // _BM,),
        in_specs=[
            pl.BlockSpec((NC, _BM, HH), lambda i: (0, i, 0)),
            pl.BlockSpec((N_CLASSES, HIDDEN), lambda i: (0, 0)),
            pl.BlockSpec((1, N_CLASSES), lambda i: (0, 0)),
        ],
        out_specs=pl.BlockSpec((_BM, N_CLASSES), lambda i: (i, 0)),
        out_shape=jax.ShapeDtypeStruct((N_PAD, N_CLASSES), jnp.float32),
    )(p, W2, b2.reshape(1, N_CLASSES))


def kernel(x, edge_index, edge_weight, W1, b1, W2, b2):
    row = edge_index[0].astype(jnp.int32)
    col = edge_index[1].astype(jnp.int32)
    w = edge_weight.astype(jnp.float32)
    pad = E_PAD - N_EDGES
    row = jnp.pad(row, (0, pad))
    col = jnp.pad(col, (0, pad))
    w = jnp.pad(w, (0, pad))
    b1h = b1.reshape(NC, 1, HH)

    xwh = _matmul_xw1(x, W1)
    p = _sc_gcn(xwh, col, row, w, b1h)
    return _final(p, W2, b2)[:N_NODES]


# speedup vs baseline: 1.2742x; 1.2742x over previous
import functools

import jax
import jax.numpy as jnp
from jax import lax
from jax.experimental import pallas as pl
from jax.experimental.pallas import tpu as pltpu
from jax.experimental.pallas import tpu_sc as plsc

N_NODES = 10000
N_EDGES = 320000
D_FEAT = 128
HIDDEN = 32
N_CLASSES = 40

NC = 2
NS = 16
L = 16
HH = HIDDEN // NC

CH = 128
NBUF = 4
HALF_CH = 80
HALF_E = HALF_CH * CH
N_HALVES = 2
T_CH = HALF_CH * N_HALVES
T_EDGES = T_CH * CH
E_PAD = NS * T_EDGES
N_PAD = 10240
ROWS_PER_SUB = N_PAD // NS


def _make_sc_gcn():
    mesh = plsc.VectorSubcoreMesh(core_axis_name="c", subcore_axis_name="s")

    @functools.partial(
        pl.kernel,
        out_type=jax.ShapeDtypeStruct((NC, N_PAD, HH), jnp.float32),
        mesh=mesh,
        compiler_params=pltpu.CompilerParams(use_tc_tiling_on_sc=False),
        scratch_types=[
            pltpu.VMEM((HALF_E,), jnp.int32),
            pltpu.VMEM((HALF_CH, CH), jnp.int32),
            pltpu.VMEM((HALF_E,), jnp.float32),
            [pltpu.VMEM((CH, HH), jnp.float32) for _ in range(NBUF)],
            pltpu.VMEM((1, HH), jnp.float32),
            [pltpu.SemaphoreType.DMA for _ in range(NBUF)],
            pltpu.SemaphoreType.DMA,
            pltpu.VMEM_SHARED((N_PAD, HH), jnp.float32),
            pltpu.VMEM_SHARED((N_PAD, HH), jnp.float32),
        ],
    )
    def sc_gcn(feat_hbm, edge_hbm, w_hbm, b1_hbm, out_hbm,
               col_v, row_v, w_v, bufs, b1_v, gsems, ssem, acc1, acc2):
        c = lax.axis_index("c")
        s = lax.axis_index("s")
        row_base = s * ROWS_PER_SUB

        pltpu.sync_copy(feat_hbm.at[c, pl.ds(row_base, ROWS_PER_SUB)],
                        acc2.at[pl.ds(row_base, ROWS_PER_SUB)])
        pltpu.sync_copy(b1_hbm.at[c], b1_v)
        bufs[0][...] = jnp.zeros((CH, HH), jnp.float32)

        def zero_into(dst):
            def zbody(z, carry):
                pltpu.sync_copy(bufs[0], dst.at[pl.ds(row_base + z * CH, CH)])
                return carry
            lax.fori_loop(0, ROWS_PER_SUB // CH, zbody, 0)

        zero_into(acc1)
        plsc.subcore_barrier()

        LOOKAHEAD = NBUF - 1

        def spmm(src, dst):
            def half_body(h, carry):
                ebase = s * T_EDGES + h * HALF_E
                pltpu.sync_copy(edge_hbm.at[1, pl.ds(ebase, HALF_E)], col_v)
                pltpu.sync_copy(w_hbm.at[pl.ds(ebase, HALF_E)], w_v)

                def rstart(i, carry2):
                    pltpu.async_copy(edge_hbm.at[0, pl.ds(ebase + i * CH, CH)],
                                     row_v.at[i], ssem)
                    return carry2
                lax.fori_loop(0, HALF_CH, rstart, 0)

                def rwait(i, carry2):
                    pltpu.make_async_copy(
                        edge_hbm.at[0, pl.ds(ebase + i * CH, CH)],
                        row_v.at[i], ssem).wait()
                    return carry2
                lax.fori_loop(0, HALF_CH, rwait, 0)

                def gather_start(lc, j):
                    pltpu.async_copy(src.at[col_v.at[pl.ds(lc * CH, CH)]],
                                     bufs[j], gsems[j])

                def gather_wait(lc, j):
                    pltpu.make_async_copy(src.at[col_v.at[pl.ds(lc * CH, CH)]],
                                          bufs[j], gsems[j]).wait()

                for pj in range(LOOKAHEAD):
                    gather_start(pj, pj)

                def body(k, carry2):
                    for j in range(NBUF):
                        lc = k * NBUF + j
                        gather_wait(lc, j)
                        for gq in range(CH // L):
                            wv = w_v[pl.ds(lc * CH + gq * L, L)]
                            for ll in range(L):
                                r = gq * L + ll
                                spl = jnp.broadcast_to(wv[ll], (L,))
                                bufs[j][r, :] = bufs[j][r, :] * spl
                        pltpu.sync_copy(bufs[j], dst.at[row_v.at[lc]],
                                        add=True)
                        lc2 = lc + LOOKAHEAD

                        @pl.when(lc2 < HALF_CH)
                        def _():
                            gather_start(lc2, (j + LOOKAHEAD) % NBUF)
                    return carry2

                lax.fori_loop(0, HALF_CH // NBUF, body, 0)
                return carry

            lax.fori_loop(0, N_HALVES, half_body, 0)

        spmm(acc2, acc1)
        plsc.subcore_barrier()

        b1exp = jnp.broadcast_to(b1_v[...], (CH, HH))

        def relu_body(z, carry):
            sl = pl.ds(row_base + z * CH, CH)
            pltpu.sync_copy(acc1.at[sl], bufs[1])
            bufs[1][...] = jnp.maximum(bufs[1][...] + b1exp, 0.0)
            pltpu.sync_copy(bufs[1], acc1.at[sl])
            return carry

        lax.fori_loop(0, ROWS_PER_SUB // CH, relu_body, 0)
        bufs[0][...] = jnp.zeros((CH, HH), jnp.float32)
        zero_into(acc2)
        plsc.subcore_barrier()

        spmm(acc1, acc2)
        plsc.subcore_barrier()

        pltpu.sync_copy(acc2.at[pl.ds(row_base, ROWS_PER_SUB)],
                        out_hbm.at[c, pl.ds(row_base, ROWS_PER_SUB)])

    return sc_gcn


_sc_gcn = _make_sc_gcn()


_BMX = 2000
_BM = 2048


def _mm1_body(x_ref, w_ref, o_ref):
    o_ref[0] = lax.dot_general(x_ref[...], w_ref[0],
                               (((1,), (1,)), ((), ())),
                               preferred_element_type=jnp.float32)


def _matmul_xw1(x, W1):
    return pl.pallas_call(
        _mm1_body,
        grid=(N_NODES // _BMX, NC),
        in_specs=[
            pl.BlockSpec((_BMX, D_FEAT), lambda i, c: (i, 0)),
            pl.BlockSpec((1, HH, D_FEAT), lambda i, c: (c, 0, 0)),
        ],
        out_specs=pl.BlockSpec((1, _BMX, HH), lambda i, c: (c, i, 0)),
        out_shape=jax.ShapeDtypeStruct((NC, N_PAD, HH), jnp.float32),
    )(x, W1.reshape(NC, HH, D_FEAT))


def _final_body(p_ref, w_ref, b_ref, o_ref):
    h = jnp.concatenate([p_ref[0], p_ref[1]], axis=1)
    o_ref[...] = lax.dot_general(h, w_ref[...], (((1,), (1,)), ((), ())),
                                 preferred_element_type=jnp.float32) + b_ref[...]


def _final(p, W2, b2):
    return pl.pallas_call(
        _final_body,
        grid=(N_PAD // _BM,),
        in_specs=[
            pl.BlockSpec((NC, _BM, HH), lambda i: (0, i, 0)),
            pl.BlockSpec((N_CLASSES, HIDDEN), lambda i: (0, 0)),
            pl.BlockSpec((1, N_CLASSES), lambda i: (0, 0)),
        ],
        out_specs=pl.BlockSpec((_BM, N_CLASSES), lambda i: (i, 0)),
        out_shape=jax.ShapeDtypeStruct((N_PAD, N_CLASSES), jnp.float32),
    )(p, W2, b2.reshape(1, N_CLASSES))


def kernel(x, edge_index, edge_weight, W1, b1, W2, b2):
    pad = E_PAD - N_EDGES
    edges = jnp.pad(edge_index.astype(jnp.int32), ((0, 0), (0, pad)))
    w = jnp.pad(edge_weight.astype(jnp.float32), (0, pad))
    b1h = b1.reshape(NC, 1, HH)

    xwh = _matmul_xw1(x, W1)
    p = _sc_gcn(xwh, edges, w, b1h)
    return _final(p, W2, b2)[:N_NODES]


# speedup vs baseline: 1.4833x; 1.1640x over previous
import functools

import jax
import jax.numpy as jnp
from jax import lax
from jax.experimental import pallas as pl
from jax.experimental.pallas import tpu as pltpu
from jax.experimental.pallas import tpu_sc as plsc

N_NODES = 10000
N_EDGES = 320000
D_FEAT = 128
HIDDEN = 32
N_CLASSES = 40

NC = 2
NS = 16
L = 16
HH = HIDDEN // NC

CH = 128
NBUF = 4
HALF_CH = 80
HALF_E = HALF_CH * CH
N_HALVES = 2
T_CH = HALF_CH * N_HALVES
T_EDGES = T_CH * CH
E_PAD = NS * T_EDGES
N_PAD = 10240
ROWS_PER_SUB = N_PAD // NS


def _make_sc_gcn():
    mesh = plsc.VectorSubcoreMesh(core_axis_name="c", subcore_axis_name="s")

    @functools.partial(
        pl.kernel,
        out_type=jax.ShapeDtypeStruct((NC, N_PAD, HH), jnp.float32),
        mesh=mesh,
        compiler_params=pltpu.CompilerParams(use_tc_tiling_on_sc=False),
        scratch_types=[
            pltpu.VMEM((HALF_E,), jnp.int32),
            pltpu.VMEM((HALF_CH, CH), jnp.int32),
            pltpu.VMEM((HALF_E,), jnp.float32),
            [pltpu.VMEM((CH, HH), jnp.float32) for _ in range(NBUF)],
            pltpu.VMEM((1, HH), jnp.float32),
            [pltpu.SemaphoreType.DMA for _ in range(NBUF)],
            [pltpu.SemaphoreType.DMA for _ in range(NBUF)],
            pltpu.SemaphoreType.DMA,
            pltpu.VMEM_SHARED((N_PAD, HH), jnp.float32),
            pltpu.VMEM_SHARED((N_PAD, HH), jnp.float32),
        ],
    )
    def sc_gcn(feat_hbm, edge_hbm, w_hbm, b1_hbm, out_hbm,
               col_v, row_v, w_v, bufs, b1_v, gsems, ssems, ssem, acc1, acc2):
        c = lax.axis_index("c")
        s = lax.axis_index("s")
        row_base = s * ROWS_PER_SUB

        pltpu.sync_copy(feat_hbm.at[c, pl.ds(row_base, ROWS_PER_SUB)],
                        acc2.at[pl.ds(row_base, ROWS_PER_SUB)])
        pltpu.sync_copy(b1_hbm.at[c], b1_v)
        bufs[0][...] = jnp.zeros((CH, HH), jnp.float32)

        def zero_into(dst):
            def zbody(z, carry):
                pltpu.sync_copy(bufs[0], dst.at[pl.ds(row_base + z * CH, CH)])
                return carry
            lax.fori_loop(0, ROWS_PER_SUB // CH, zbody, 0)

        zero_into(acc1)
        plsc.subcore_barrier()

        LOOKAHEAD = NBUF - 1

        def spmm(src, dst):
            def half_body(h, carry):
                ebase = s * T_EDGES + h * HALF_E
                pltpu.sync_copy(edge_hbm.at[1, pl.ds(ebase, HALF_E)], col_v)
                pltpu.sync_copy(w_hbm.at[pl.ds(ebase, HALF_E)], w_v)

                def rstart(i, carry2):
                    pltpu.async_copy(edge_hbm.at[0, pl.ds(ebase + i * CH, CH)],
                                     row_v.at[i], ssem)
                    return carry2
                lax.fori_loop(0, HALF_CH, rstart, 0)

                def rwait(i, carry2):
                    pltpu.make_async_copy(
                        edge_hbm.at[0, pl.ds(ebase + i * CH, CH)],
                        row_v.at[i], ssem).wait()
                    return carry2
                lax.fori_loop(0, HALF_CH, rwait, 0)

                def gather_start(lc, j):
                    pltpu.async_copy(src.at[col_v.at[pl.ds(lc * CH, CH)]],
                                     bufs[j], gsems[j])

                def gather_wait(lc, j):
                    pltpu.make_async_copy(src.at[col_v.at[pl.ds(lc * CH, CH)]],
                                          bufs[j], gsems[j]).wait()

                for pj in range(LOOKAHEAD):
                    gather_start(pj, pj)

                def body(k, carry2):
                    for j in range(NBUF):
                        lc = k * NBUF + j
                        gather_wait(lc, j)
                        for gq in range(CH // L):
                            wv = w_v[pl.ds(lc * CH + gq * L, L)]
                            for ll in range(L):
                                r = gq * L + ll
                                spl = jnp.broadcast_to(wv[ll], (L,))
                                bufs[j][r, :] = bufs[j][r, :] * spl
                        pltpu.async_copy(bufs[j], dst.at[row_v.at[lc]],
                                         ssems[j], add=True)
                        lc2 = lc + LOOKAHEAD
                        j2 = (j + LOOKAHEAD) % NBUF

                        @pl.when(lc2 < HALF_CH)
                        def _():
                            @pl.when(lc >= 1)
                            def _():
                                pltpu.make_async_copy(
                                    bufs[j2], dst.at[row_v.at[lc - 1]],
                                    ssems[j2]).wait()
                            gather_start(lc2, j2)
                    return carry2

                lax.fori_loop(0, HALF_CH // NBUF, body, 0)
                for dj in range(NBUF):
                    dlc = HALF_CH - NBUF + dj
                    pltpu.make_async_copy(bufs[dlc % NBUF],
                                          dst.at[row_v.at[dlc]],
                                          ssems[dlc % NBUF]).wait()
                return carry

            lax.fori_loop(0, N_HALVES, half_body, 0)

        spmm(acc2, acc1)
        plsc.subcore_barrier()

        b1exp = jnp.broadcast_to(b1_v[...], (CH, HH))

        def relu_body(z, carry):
            sl = pl.ds(row_base + z * CH, CH)
            pltpu.sync_copy(acc1.at[sl], bufs[1])
            bufs[1][...] = jnp.maximum(bufs[1][...] + b1exp, 0.0)
            pltpu.sync_copy(bufs[1], acc1.at[sl])
            return carry

        lax.fori_loop(0, ROWS_PER_SUB // CH, relu_body, 0)
        bufs[0][...] = jnp.zeros((CH, HH), jnp.float32)
        zero_into(acc2)
        plsc.subcore_barrier()

        spmm(acc1, acc2)
        plsc.subcore_barrier()

        pltpu.sync_copy(acc2.at[pl.ds(row_base, ROWS_PER_SUB)],
                        out_hbm.at[c, pl.ds(row_base, ROWS_PER_SUB)])

    return sc_gcn


_sc_gcn = _make_sc_gcn()


_BMX = 2000
_BM = 2048


def _mm1_body(x_ref, w_ref, o_ref):
    o_ref[0] = lax.dot_general(x_ref[...], w_ref[0],
                               (((1,), (1,)), ((), ())),
                               preferred_element_type=jnp.float32)


def _matmul_xw1(x, W1):
    return pl.pallas_call(
        _mm1_body,
        grid=(N_NODES // _BMX, NC),
        in_specs=[
            pl.BlockSpec((_BMX, D_FEAT), lambda i, c: (i, 0)),
            pl.BlockSpec((1, HH, D_FEAT), lambda i, c: (c, 0, 0)),
        ],
        out_specs=pl.BlockSpec((1, _BMX, HH), lambda i, c: (c, i, 0)),
        out_shape=jax.ShapeDtypeStruct((NC, N_PAD, HH), jnp.float32),
    )(x, W1.reshape(NC, HH, D_FEAT))


def _final_body(p_ref, w_ref, b_ref, o_ref):
    h = jnp.concatenate([p_ref[0], p_ref[1]], axis=1)
    o_ref[...] = lax.dot_general(h, w_ref[...], (((1,), (1,)), ((), ())),
                                 preferred_element_type=jnp.float32) + b_ref[...]


def _final(p, W2, b2):
    return pl.pallas_call(
        _final_body,
        grid=(N_PAD // _BM,),
        in_specs=[
            pl.BlockSpec((NC, _BM, HH), lambda i: (0, i, 0)),
            pl.BlockSpec((N_CLASSES, HIDDEN), lambda i: (0, 0)),
            pl.BlockSpec((1, N_CLASSES), lambda i: (0, 0)),
        ],
        out_specs=pl.BlockSpec((_BM, N_CLASSES), lambda i: (i, 0)),
        out_shape=jax.ShapeDtypeStruct((N_PAD, N_CLASSES), jnp.float32),
    )(p, W2, b2.reshape(1, N_CLASSES))


def kernel(x, edge_index, edge_weight, W1, b1, W2, b2):
    pad = E_PAD - N_EDGES
    edges = jnp.pad(edge_index.astype(jnp.int32), ((0, 0), (0, pad)))
    w = jnp.pad(edge_weight.astype(jnp.float32), (0, pad))
    b1h = b1.reshape(NC, 1, HH)

    xwh = _matmul_xw1(x, W1)
    p = _sc_gcn(xwh, edges, w, b1h)
    return _final(p, W2, b2)[:N_NODES]


# speedup vs baseline: 1.5275x; 1.0298x over previous
import functools

import jax
import jax.numpy as jnp
from jax import lax
from jax.experimental import pallas as pl
from jax.experimental.pallas import tpu as pltpu
from jax.experimental.pallas import tpu_sc as plsc

N_NODES = 10000
N_EDGES = 320000
D_FEAT = 128
HIDDEN = 32
N_CLASSES = 40

NC = 2
NS = 16
L = 16
HH = HIDDEN // NC

CH = 128
NBUF = 4
HALF_CH = 80
HALF_E = HALF_CH * CH
N_HALVES = 2
T_CH = HALF_CH * N_HALVES
T_EDGES = T_CH * CH
E_PAD = NS * T_EDGES
N_PAD = 10240
ROWS_PER_SUB = N_PAD // NS


def _make_sc_gcn():
    mesh = plsc.VectorSubcoreMesh(core_axis_name="c", subcore_axis_name="s")

    @functools.partial(
        pl.kernel,
        out_type=jax.ShapeDtypeStruct((NC, N_PAD, HH), jnp.float32),
        mesh=mesh,
        compiler_params=pltpu.CompilerParams(use_tc_tiling_on_sc=False),
        scratch_types=[
            pltpu.VMEM((HALF_E,), jnp.int32),
            pltpu.VMEM((HALF_CH, CH), jnp.int32),
            pltpu.VMEM((HALF_E,), jnp.float32),
            [pltpu.VMEM((CH, HH), jnp.float32) for _ in range(NBUF)],
            pltpu.VMEM((1, HH), jnp.float32),
            [pltpu.SemaphoreType.DMA for _ in range(NBUF)],
            [pltpu.SemaphoreType.DMA for _ in range(NBUF)],
            pltpu.SemaphoreType.DMA,
            pltpu.VMEM_SHARED((N_PAD, HH), jnp.float32),
            pltpu.VMEM_SHARED((N_PAD, HH), jnp.float32),
        ],
    )
    def sc_gcn(feat_hbm, edge_hbm, w_hbm, b1_hbm, out_hbm,
               col_v, row_v, w_v, bufs, b1_v, gsems, ssems, ssem, acc1, acc2):
        c = lax.axis_index("c")
        s = lax.axis_index("s")
        row_base = s * ROWS_PER_SUB

        pltpu.sync_copy(feat_hbm.at[c, pl.ds(row_base, ROWS_PER_SUB)],
                        acc2.at[pl.ds(row_base, ROWS_PER_SUB)])
        pltpu.sync_copy(b1_hbm.at[c], b1_v)
        bufs[0][...] = jnp.zeros((CH, HH), jnp.float32)

        def zero_into(dst):
            def zbody(z, carry):
                pltpu.sync_copy(bufs[0], dst.at[pl.ds(row_base + z * CH, CH)])
                return carry
            lax.fori_loop(0, ROWS_PER_SUB // CH, zbody, 0)

        zero_into(acc1)
        plsc.subcore_barrier()

        LOOKAHEAD = NBUF - 1

        def spmm(src, dst):
            def half_body(h, carry):
                ebase = s * T_EDGES + h * HALF_E
                pltpu.sync_copy(edge_hbm.at[1, pl.ds(ebase, HALF_E)], col_v)
                pltpu.sync_copy(w_hbm.at[pl.ds(ebase, HALF_E)], w_v)

                def rstart(i, carry2):
                    pltpu.async_copy(edge_hbm.at[0, pl.ds(ebase + i * CH, CH)],
                                     row_v.at[i], ssem)
                    return carry2
                lax.fori_loop(0, HALF_CH, rstart, 0)

                def rwait(i, carry2):
                    pltpu.make_async_copy(
                        edge_hbm.at[0, pl.ds(ebase + i * CH, CH)],
                        row_v.at[i], ssem).wait()
                    return carry2
                lax.fori_loop(0, HALF_CH, rwait, 0)

                def gather_start(lc, j):
                    pltpu.async_copy(src.at[col_v.at[pl.ds(lc * CH, CH)]],
                                     bufs[j], gsems[j])

                def gather_wait(lc, j):
                    pltpu.make_async_copy(src.at[col_v.at[pl.ds(lc * CH, CH)]],
                                          bufs[j], gsems[j]).wait()

                for pj in range(LOOKAHEAD):
                    gather_start(pj, pj)

                def body(k, carry2):
                    for j in range(NBUF):
                        lc = k * NBUF + j
                        gather_wait(lc, j)
                        for gq in range(CH // L):
                            wv = w_v[pl.ds(lc * CH + gq * L, L)]
                            for ll in range(L):
                                r = gq * L + ll
                                spl = jnp.broadcast_to(wv[ll], (L,))
                                bufs[j][r, :] = bufs[j][r, :] * spl
                        pltpu.async_copy(bufs[j], dst.at[row_v.at[lc]],
                                         ssems[j], add=True)
                        lc2 = lc + LOOKAHEAD
                        j2 = (j + LOOKAHEAD) % NBUF

                        @pl.when(lc2 < HALF_CH)
                        def _():
                            @pl.when(lc >= 1)
                            def _():
                                pltpu.make_async_copy(
                                    bufs[j2], dst.at[row_v.at[lc - 1]],
                                    ssems[j2]).wait()
                            gather_start(lc2, j2)
                    return carry2

                lax.fori_loop(0, HALF_CH // NBUF, body, 0)
                for dj in range(NBUF):
                    dlc = HALF_CH - NBUF + dj
                    pltpu.make_async_copy(bufs[dlc % NBUF],
                                          dst.at[row_v.at[dlc]],
                                          ssems[dlc % NBUF]).wait()
                return carry

            lax.fori_loop(0, N_HALVES, half_body, 0)

        spmm(acc2, acc1)
        plsc.subcore_barrier()

        b1exp = jnp.broadcast_to(b1_v[...], (CH, HH))

        def relu_body(z, carry):
            sl = pl.ds(row_base + z * CH, CH)
            pltpu.sync_copy(acc1.at[sl], bufs[1])
            bufs[1][...] = jnp.maximum(bufs[1][...] + b1exp, 0.0)
            pltpu.sync_copy(bufs[1], acc1.at[sl])
            return carry

        lax.fori_loop(0, ROWS_PER_SUB // CH, relu_body, 0)
        bufs[0][...] = jnp.zeros((CH, HH), jnp.float32)
        zero_into(acc2)
        plsc.subcore_barrier()

        spmm(acc1, acc2)
        plsc.subcore_barrier()

        pltpu.sync_copy(acc2.at[pl.ds(row_base, ROWS_PER_SUB)],
                        out_hbm.at[c, pl.ds(row_base, ROWS_PER_SUB)])

    return sc_gcn


_sc_gcn = _make_sc_gcn()


_BMX = 2000
_BM = 2048


def _mm1_body(x_ref, w_ref, o_ref):
    xb = x_ref[...]
    for c in range(NC):
        o_ref[c] = lax.dot_general(xb, w_ref[c],
                                   (((1,), (1,)), ((), ())),
                                   preferred_element_type=jnp.float32)


def _matmul_xw1(x, W1):
    return pl.pallas_call(
        _mm1_body,
        grid=(N_NODES // _BMX,),
        in_specs=[
            pl.BlockSpec((_BMX, D_FEAT), lambda i: (i, 0)),
            pl.BlockSpec((NC, HH, D_FEAT), lambda i: (0, 0, 0)),
        ],
        out_specs=pl.BlockSpec((NC, _BMX, HH), lambda i: (0, i, 0)),
        out_shape=jax.ShapeDtypeStruct((NC, N_PAD, HH), jnp.float32),
    )(x, W1.reshape(NC, HH, D_FEAT))


def _final_body(p_ref, w_ref, b_ref, o_ref):
    h = jnp.concatenate([p_ref[0], p_ref[1]], axis=1)
    o_ref[...] = lax.dot_general(h, w_ref[...], (((1,), (1,)), ((), ())),
                                 preferred_element_type=jnp.float32) + b_ref[...]


def _final(p, W2, b2):
    return pl.pallas_call(
        _final_body,
        grid=(N_NODES // _BMX,),
        in_specs=[
            pl.BlockSpec((NC, _BMX, HH), lambda i: (0, i, 0)),
            pl.BlockSpec((N_CLASSES, HIDDEN), lambda i: (0, 0)),
            pl.BlockSpec((1, N_CLASSES), lambda i: (0, 0)),
        ],
        out_specs=pl.BlockSpec((_BMX, N_CLASSES), lambda i: (i, 0)),
        out_shape=jax.ShapeDtypeStruct((N_NODES, N_CLASSES), jnp.float32),
    )(p, W2, b2.reshape(1, N_CLASSES))


def kernel(x, edge_index, edge_weight, W1, b1, W2, b2):
    pad = E_PAD - N_EDGES
    edges = jnp.pad(edge_index.astype(jnp.int32), ((0, 0), (0, pad)))
    w = jnp.pad(edge_weight.astype(jnp.float32), (0, pad))
    b1h = b1.reshape(NC, 1, HH)

    xwh = _matmul_xw1(x, W1)
    p = _sc_gcn(xwh, edges, w, b1h)
    return _final(p, W2, b2)
